# baseline (device time: 88179 ns/iter reference)
import jax
import jax.numpy as jnp
from jax import lax
from jax.experimental import pallas as pl
from jax.experimental.pallas import tpu as pltpu

N_DEV = 4


def kernel(x, w_mat):
    m, k = x.shape
    _, n = w_mat.shape
    m_chunk = m // N_DEV

    def body(x_ref, w_ref, out_ref, p_ref, send_buf, recv_buf, send_sems, recv_sems):
        my = lax.axis_index("i")
        left = (my - 1) % N_DEV
        right = (my + 1) % N_DEV

        barrier_sem = pltpu.get_barrier_semaphore()
        for nbr in [left, right]:
            pl.semaphore_signal(
                barrier_sem, inc=1,
                device_id=(nbr,), device_id_type=pl.DeviceIdType.MESH,
            )
        pl.semaphore_wait(barrier_sem, 2)

        p_ref[:, :] = jnp.dot(
            x_ref[:, :].astype(jnp.bfloat16),
            w_ref[:, :].astype(jnp.bfloat16),
            preferred_element_type=jnp.float32,
        )

        def chunk(c):
            return p_ref[pl.ds(c * m_chunk, m_chunk), :]

        send_buf[0, :, :] = chunk((my - 1) % N_DEV).astype(jnp.bfloat16)
        for h in range(N_DEV - 1):
            rdma = pltpu.make_async_remote_copy(
                src_ref=send_buf.at[h],
                dst_ref=recv_buf.at[h],
                send_sem=send_sems.at[h],
                recv_sem=recv_sems.at[h],
                device_id=(right,),
                device_id_type=pl.DeviceIdType.MESH,
            )
            rdma.start()
            rdma.wait()

            c = (my - 2 - h) % N_DEV
            acc = recv_buf[h, :, :].astype(jnp.float32) + chunk(c)
            if h < N_DEV - 2:
                send_buf[h + 1, :, :] = acc.astype(jnp.bfloat16)
            else:
                out_ref[:, :] = acc

    return pl.pallas_call(
        body,
        out_shape=jax.ShapeDtypeStruct((m_chunk, n), jnp.float32),
        in_specs=[
            pl.BlockSpec(memory_space=pltpu.VMEM),
            pl.BlockSpec(memory_space=pltpu.VMEM),
        ],
        out_specs=pl.BlockSpec(memory_space=pltpu.VMEM),
        scratch_shapes=[
            pltpu.VMEM((m, n), jnp.float32),
            pltpu.VMEM((N_DEV - 1, m_chunk, n), jnp.bfloat16),
            pltpu.VMEM((N_DEV - 1, m_chunk, n), jnp.bfloat16),
            pltpu.SemaphoreType.DMA((N_DEV - 1,)),
            pltpu.SemaphoreType.DMA((N_DEV - 1,)),
        ],
        compiler_params=pltpu.CompilerParams(collective_id=0),
    )(x, w_mat)


# device time: 52135 ns/iter; 1.6914x vs baseline; 1.6914x over previous
import jax
import jax.numpy as jnp
from jax import lax
from jax.experimental import pallas as pl
from jax.experimental.pallas import tpu as pltpu

N_DEV = 4


def kernel(x, w_mat):
    m, k = x.shape
    _, n = w_mat.shape
    m_chunk = m // N_DEV
    n2 = n // 2

    def body(x_ref, w_ref, out_ref, p_ref,
             cw_send, cw_recv, ccw_send, ccw_recv,
             cw_ssem, cw_rsem, ccw_ssem, ccw_rsem):
        my = lax.axis_index("i")
        left = (my - 1) % N_DEV
        right = (my + 1) % N_DEV

        barrier_sem = pltpu.get_barrier_semaphore()
        for nbr in [left, right]:
            pl.semaphore_signal(
                barrier_sem, inc=1,
                device_id=(nbr,), device_id_type=pl.DeviceIdType.MESH,
            )
        pl.semaphore_wait(barrier_sem, 2)

        wb = w_ref[:, :].astype(jnp.bfloat16)

        def compute_chunk(c):
            p_ref[pl.ds(c * m_chunk, m_chunk), :] = jnp.dot(
                x_ref[pl.ds(c * m_chunk, m_chunk), :].astype(jnp.bfloat16), wb,
                preferred_element_type=jnp.float32,
            )

        def chunk_l(c):
            return p_ref[pl.ds(c * m_chunk, m_chunk), :n2]

        def chunk_r(c):
            return p_ref[pl.ds(c * m_chunk, m_chunk), n2:]

        compute_chunk((my - 1) % N_DEV)
        compute_chunk((my + 1) % N_DEV)
        cw_send[0, :, :] = chunk_l((my - 1) % N_DEV).astype(jnp.bfloat16)
        ccw_send[0, :, :] = chunk_r((my + 1) % N_DEV).astype(jnp.bfloat16)

        for h in range(N_DEV - 1):
            cw = pltpu.make_async_remote_copy(
                src_ref=cw_send.at[h], dst_ref=cw_recv.at[h],
                send_sem=cw_ssem.at[h], recv_sem=cw_rsem.at[h],
                device_id=(right,), device_id_type=pl.DeviceIdType.MESH,
            )
            ccw = pltpu.make_async_remote_copy(
                src_ref=ccw_send.at[h], dst_ref=ccw_recv.at[h],
                send_sem=ccw_ssem.at[h], recv_sem=ccw_rsem.at[h],
                device_id=(left,), device_id_type=pl.DeviceIdType.MESH,
            )
            cw.start()
            ccw.start()

            if h == 0:
                compute_chunk((my + 2) % N_DEV)
                compute_chunk(my)

            cw.wait()
            ccw.wait()

            c_cw = (my - 2 - h) % N_DEV
            c_ccw = (my + 2 + h) % N_DEV
            acc_cw = cw_recv[h, :, :].astype(jnp.float32) + chunk_l(c_cw)
            acc_ccw = ccw_recv[h, :, :].astype(jnp.float32) + chunk_r(c_ccw)
            if h < N_DEV - 2:
                cw_send[h + 1, :, :] = acc_cw.astype(jnp.bfloat16)
                ccw_send[h + 1, :, :] = acc_ccw.astype(jnp.bfloat16)
            else:
                out_ref[:, :n2] = acc_cw
                out_ref[:, n2:] = acc_ccw

    return pl.pallas_call(
        body,
        out_shape=jax.ShapeDtypeStruct((m_chunk, n), jnp.float32),
        in_specs=[
            pl.BlockSpec(memory_space=pltpu.VMEM),
            pl.BlockSpec(memory_space=pltpu.VMEM),
        ],
        out_specs=pl.BlockSpec(memory_space=pltpu.VMEM),
        scratch_shapes=[
            pltpu.VMEM((m, n), jnp.float32),
            pltpu.VMEM((N_DEV - 1, m_chunk, n2), jnp.bfloat16),
            pltpu.VMEM((N_DEV - 1, m_chunk, n2), jnp.bfloat16),
            pltpu.VMEM((N_DEV - 1, m_chunk, n2), jnp.bfloat16),
            pltpu.VMEM((N_DEV - 1, m_chunk, n2), jnp.bfloat16),
            pltpu.SemaphoreType.DMA((N_DEV - 1,)),
            pltpu.SemaphoreType.DMA((N_DEV - 1,)),
            pltpu.SemaphoreType.DMA((N_DEV - 1,)),
            pltpu.SemaphoreType.DMA((N_DEV - 1,)),
        ],
        compiler_params=pltpu.CompilerParams(collective_id=0),
    )(x, w_mat)


# device time: 50763 ns/iter; 1.7371x vs baseline; 1.0270x over previous
import jax
import jax.numpy as jnp
from jax import lax
from jax.experimental import pallas as pl
from jax.experimental.pallas import tpu as pltpu

N_DEV = 4


def kernel(x, w_mat):
    m, k = x.shape
    _, n = w_mat.shape
    m_chunk = m // N_DEV
    n2 = n // 2

    def body(x_ref, w_ref, out_ref, p_ref,
             cw_send, cw_recv, ccw_send, ccw_recv,
             cw_ssem, cw_rsem, ccw_ssem, ccw_rsem):
        my = lax.axis_index("i")
        left = (my - 1) % N_DEV
        right = (my + 1) % N_DEV

        barrier_sem = pltpu.get_barrier_semaphore()
        for nbr in [left, right]:
            pl.semaphore_signal(
                barrier_sem, inc=1,
                device_id=(nbr,), device_id_type=pl.DeviceIdType.MESH,
            )
        pl.semaphore_wait(barrier_sem, 2)

        def xrows(c):
            return x_ref[pl.ds(c * m_chunk, m_chunk), :].astype(jnp.bfloat16)

        def make_hop(h):
            cw = pltpu.make_async_remote_copy(
                src_ref=cw_send.at[h], dst_ref=cw_recv.at[h],
                send_sem=cw_ssem.at[h], recv_sem=cw_rsem.at[h],
                device_id=(right,), device_id_type=pl.DeviceIdType.MESH,
            )
            ccw = pltpu.make_async_remote_copy(
                src_ref=ccw_send.at[h], dst_ref=ccw_recv.at[h],
                send_sem=ccw_ssem.at[h], recv_sem=ccw_rsem.at[h],
                device_id=(left,), device_id_type=pl.DeviceIdType.MESH,
            )
            return cw, ccw

        wl = w_ref[:, :n2].astype(jnp.bfloat16)
        wr = w_ref[:, n2:].astype(jnp.bfloat16)
        cw_send[0, :, :] = jnp.dot(
            xrows((my - 1) % N_DEV), wl, preferred_element_type=jnp.float32,
        ).astype(jnp.bfloat16)
        ccw_send[0, :, :] = jnp.dot(
            xrows((my + 1) % N_DEV), wr, preferred_element_type=jnp.float32,
        ).astype(jnp.bfloat16)

        hops = [make_hop(0)]
        hops[0][0].start()
        hops[0][1].start()

        c2 = (my + 2) % N_DEV
        p_ref[pl.ds(c2 * m_chunk, m_chunk), :] = jnp.dot(
            xrows(c2), jnp.concatenate([wl, wr], axis=1),
            preferred_element_type=jnp.float32,
        )
        p_ref[pl.ds(((my + 1) % N_DEV) * m_chunk, m_chunk), :n2] = jnp.dot(
            xrows((my + 1) % N_DEV), wl, preferred_element_type=jnp.float32,
        )
        p_ref[pl.ds(((my - 1) % N_DEV) * m_chunk, m_chunk), n2:] = jnp.dot(
            xrows((my - 1) % N_DEV), wr, preferred_element_type=jnp.float32,
        )
        p_ref[pl.ds(my * m_chunk, m_chunk), :] = jnp.dot(
            xrows(my), jnp.concatenate([wl, wr], axis=1),
            preferred_element_type=jnp.float32,
        )

        def chunk_l(c):
            return p_ref[pl.ds(c * m_chunk, m_chunk), :n2]

        def chunk_r(c):
            return p_ref[pl.ds(c * m_chunk, m_chunk), n2:]

        for h in range(N_DEV - 1):
            cw, ccw = hops[h]
            cw.wait_recv()
            ccw.wait_recv()

            c_cw = (my - 2 - h) % N_DEV
            c_ccw = (my + 2 + h) % N_DEV
            acc_cw = cw_recv[h, :, :].astype(jnp.float32) + chunk_l(c_cw)
            acc_ccw = ccw_recv[h, :, :].astype(jnp.float32) + chunk_r(c_ccw)
            if h < N_DEV - 2:
                cw_send[h + 1, :, :] = acc_cw.astype(jnp.bfloat16)
                ccw_send[h + 1, :, :] = acc_ccw.astype(jnp.bfloat16)
                nxt = make_hop(h + 1)
                nxt[0].start()
                nxt[1].start()
                hops.append(nxt)
            else:
                out_ref[:, :n2] = acc_cw
                out_ref[:, n2:] = acc_ccw

        for cw, ccw in hops:
            cw.wait_send()
            ccw.wait_send()

    return pl.pallas_call(
        body,
        out_shape=jax.ShapeDtypeStruct((m_chunk, n), jnp.float32),
        in_specs=[
            pl.BlockSpec(memory_space=pltpu.VMEM),
            pl.BlockSpec(memory_space=pltpu.VMEM),
        ],
        out_specs=pl.BlockSpec(memory_space=pltpu.VMEM),
        scratch_shapes=[
            pltpu.VMEM((m, n), jnp.float32),
            pltpu.VMEM((N_DEV - 1, m_chunk, n2), jnp.bfloat16),
            pltpu.VMEM((N_DEV - 1, m_chunk, n2), jnp.bfloat16),
            pltpu.VMEM((N_DEV - 1, m_chunk, n2), jnp.bfloat16),
            pltpu.VMEM((N_DEV - 1, m_chunk, n2), jnp.bfloat16),
            pltpu.SemaphoreType.DMA((N_DEV - 1,)),
            pltpu.SemaphoreType.DMA((N_DEV - 1,)),
            pltpu.SemaphoreType.DMA((N_DEV - 1,)),
            pltpu.SemaphoreType.DMA((N_DEV - 1,)),
        ],
        compiler_params=pltpu.CompilerParams(collective_id=0),
    )(x, w_mat)


# device time: 46059 ns/iter; 1.9145x vs baseline; 1.1021x over previous
import jax
import jax.numpy as jnp
from jax import lax
from jax.experimental import pallas as pl
from jax.experimental.pallas import tpu as pltpu

N_DEV = 4
NSEG = 2


def kernel(x, w_mat):
    m, k = x.shape
    _, n = w_mat.shape
    m_chunk = m // N_DEV
    n2 = n // 2
    nseg = n2 // NSEG

    def body(x_ref, w_ref, out_ref, p_ref,
             cw_send, cw_recv, ccw_send, ccw_recv,
             cw_ssem, cw_rsem, ccw_ssem, ccw_rsem):
        my = lax.axis_index("i")
        left = (my - 1) % N_DEV
        right = (my + 1) % N_DEV

        barrier_sem = pltpu.get_barrier_semaphore()
        for nbr in [left, right]:
            pl.semaphore_signal(
                barrier_sem, inc=1,
                device_id=(nbr,), device_id_type=pl.DeviceIdType.MESH,
            )
        pl.semaphore_wait(barrier_sem, 2)

        def xrows(c):
            return x_ref[pl.ds(c * m_chunk, m_chunk), :].astype(jnp.bfloat16)

        wb = w_ref[:, :].astype(jnp.bfloat16)

        def make_seg(h, s, direction):
            if direction == 0:
                return pltpu.make_async_remote_copy(
                    src_ref=cw_send.at[h, s], dst_ref=cw_recv.at[h, s],
                    send_sem=cw_ssem.at[h, s], recv_sem=cw_rsem.at[h, s],
                    device_id=(right,), device_id_type=pl.DeviceIdType.MESH,
                )
            return pltpu.make_async_remote_copy(
                src_ref=ccw_send.at[h, s], dst_ref=ccw_recv.at[h, s],
                send_sem=ccw_ssem.at[h, s], recv_sem=ccw_rsem.at[h, s],
                device_id=(left,), device_id_type=pl.DeviceIdType.MESH,
            )

        def col_lo(direction, s):
            return direction * n2 + s * nseg

        def p_seg(c, direction, s):
            lo = col_lo(direction, s)
            return p_ref[pl.ds(c * m_chunk, m_chunk), lo:lo + nseg]

        rdmas = []
        x_cw = xrows((my - 1) % N_DEV)
        x_ccw = xrows((my + 1) % N_DEV)
        for s in range(NSEG):
            for direction, xv in ((0, x_cw), (1, x_ccw)):
                lo = col_lo(direction, s)
                sb = cw_send if direction == 0 else ccw_send
                sb[0, s, :, :] = jnp.dot(
                    xv, wb[:, lo:lo + nseg],
                    preferred_element_type=jnp.float32,
                ).astype(jnp.bfloat16)
                r = make_seg(0, s, direction)
                r.start()
                rdmas.append(r)

        c2 = (my + 2) % N_DEV
        p_ref[pl.ds(c2 * m_chunk, m_chunk), :] = jnp.dot(
            xrows(c2), wb, preferred_element_type=jnp.float32,
        ).astype(jnp.bfloat16)
        p_ref[pl.ds(((my + 1) % N_DEV) * m_chunk, m_chunk), :n2] = jnp.dot(
            x_ccw, wb[:, :n2], preferred_element_type=jnp.float32,
        ).astype(jnp.bfloat16)
        p_ref[pl.ds(((my - 1) % N_DEV) * m_chunk, m_chunk), n2:] = jnp.dot(
            x_cw, wb[:, n2:], preferred_element_type=jnp.float32,
        ).astype(jnp.bfloat16)
        p_ref[pl.ds(my * m_chunk, m_chunk), :] = jnp.dot(
            xrows(my), wb, preferred_element_type=jnp.float32,
        ).astype(jnp.bfloat16)

        hop_rdmas = {(0, s, d): rdmas[2 * s + d] for s in range(NSEG) for d in (0, 1)}
        for h in range(N_DEV - 1):
            c_cw = (my - 2 - h) % N_DEV
            c_ccw = (my + 2 + h) % N_DEV
            for s in range(NSEG):
                for direction, c in ((0, c_cw), (1, c_ccw)):
                    rb = cw_recv if direction == 0 else ccw_recv
                    hop_rdmas[(h, s, direction)].wait_recv()
                    acc = (
                        rb[h, s, :, :].astype(jnp.float32)
                        + p_seg(c, direction, s).astype(jnp.float32)
                    )
                    if h < N_DEV - 2:
                        sb = cw_send if direction == 0 else ccw_send
                        sb[h + 1, s, :, :] = acc.astype(jnp.bfloat16)
                        r = make_seg(h + 1, s, direction)
                        r.start()
                        hop_rdmas[(h + 1, s, direction)] = r
                        rdmas.append(r)
                    else:
                        lo = col_lo(direction, s)
                        out_ref[:, lo:lo + nseg] = acc

        for r in rdmas:
            r.wait_send()

    comm_shape = (N_DEV - 1, NSEG, m_chunk, nseg)
    sem_shape = (N_DEV - 1, NSEG)
    return pl.pallas_call(
        body,
        out_shape=jax.ShapeDtypeStruct((m_chunk, n), jnp.float32),
        in_specs=[
            pl.BlockSpec(memory_space=pltpu.VMEM),
            pl.BlockSpec(memory_space=pltpu.VMEM),
        ],
        out_specs=pl.BlockSpec(memory_space=pltpu.VMEM),
        scratch_shapes=[
            pltpu.VMEM((m, n), jnp.bfloat16),
            pltpu.VMEM(comm_shape, jnp.bfloat16),
            pltpu.VMEM(comm_shape, jnp.bfloat16),
            pltpu.VMEM(comm_shape, jnp.bfloat16),
            pltpu.VMEM(comm_shape, jnp.bfloat16),
            pltpu.SemaphoreType.DMA(sem_shape),
            pltpu.SemaphoreType.DMA(sem_shape),
            pltpu.SemaphoreType.DMA(sem_shape),
            pltpu.SemaphoreType.DMA(sem_shape),
        ],
        compiler_params=pltpu.CompilerParams(collective_id=0),
    )(x, w_mat)
